# baseline (speedup 1.0000x reference)
import functools

import numpy as np

import jax
import jax.numpy as jnp
from jax import lax
from jax.experimental import pallas as pl
from jax.experimental.pallas import tpu as pltpu
from jax.experimental.pallas import tpu_sc as plsc

N = 10000
E = 320000
D = 128
DH = D // 2

N_TC = 10240
N_PAD = 10112
NC = 2
NS = 16
CHUNK = 128
NCHUNKS = E // CHUNK
ROWS_PER_TILE = N_PAD // NS

NW = NC * NS
P1_BASE = NCHUNKS // NW
P1_EXTRA = NCHUNKS % NW
P1_SUBS = 13
P1_SUPERS = P1_BASE // P1_SUBS
P1_SUPER_E = P1_SUBS * CHUNK

BASE_CHUNKS = NCHUNKS // NS
EXTRA_CHUNKS = NCHUNKS % NS
SUBS = 12
SUPERS = BASE_CHUNKS // SUBS
SUPER_E = SUBS * CHUNK
NBH = 3
NBE = 2


def _tc_node_body(x_ref, wfc_ref, lw_ref, aw_ref,
                  hlo_ref, hhi_ref, hl_ref, s12_ref):
    x = x_ref[...]
    h = lax.dot_general(x, wfc_ref[...], (((1,), (1,)), ((), ())),
                        preferred_element_type=jnp.float32)
    hlo_ref[...] = h[:, :DH].astype(jnp.bfloat16)
    hhi_ref[...] = h[:, DH:].astype(jnp.bfloat16)
    hl_ref[...] = lax.dot_general(h, lw_ref[...], (((1,), (0,)), ((), ())),
                                  preferred_element_type=jnp.float32)
    a1 = aw_ref[:, 0:D]
    a2 = aw_ref[:, D:2 * D]
    s1 = lax.dot_general(a1, h, (((1,), (1,)), ((), ())),
                         preferred_element_type=jnp.float32)
    s2 = lax.dot_general(a2, h, (((1,), (1,)), ((), ())),
                         preferred_element_type=jnp.float32)
    s12_ref[...] = jnp.concatenate(
        [s1, s2, jnp.zeros((6, s1.shape[1]), jnp.float32)], axis=0)


def _tc_node(x_pad, W_fc, loop_w, attn_w):
    blk = 512
    grid = N_TC // blk
    return pl.pallas_call(
        _tc_node_body,
        grid=(grid,),
        in_specs=[
            pl.BlockSpec((blk, D), lambda i: (i, 0)),
            pl.BlockSpec((D, D), lambda i: (0, 0)),
            pl.BlockSpec((D, D), lambda i: (0, 0)),
            pl.BlockSpec((1, 3 * D), lambda i: (0, 0)),
        ],
        out_specs=[
            pl.BlockSpec((blk, DH), lambda i: (i, 0)),
            pl.BlockSpec((blk, DH), lambda i: (i, 0)),
            pl.BlockSpec((blk, D), lambda i: (i, 0)),
            pl.BlockSpec((8, blk), lambda i: (0, i)),
        ],
        out_shape=[
            jax.ShapeDtypeStruct((N_TC, DH), jnp.bfloat16),
            jax.ShapeDtypeStruct((N_TC, DH), jnp.bfloat16),
            jax.ShapeDtypeStruct((N_TC, D), jnp.float32),
            jax.ShapeDtypeStruct((8, N_TC), jnp.float32),
        ],
    )(x_pad, W_fc, loop_w, attn_w)


def _tc_edge_body(ea_ref, wr_ref, aw_ref, ehlo_ref, ehhi_ref, s3_ref):
    eh = lax.dot_general(ea_ref[...], wr_ref[...], (((1,), (1,)), ((), ())),
                         preferred_element_type=jnp.float32)
    ehlo_ref[...] = eh[:, :DH].astype(jnp.bfloat16)
    ehhi_ref[...] = eh[:, DH:].astype(jnp.bfloat16)
    a3 = aw_ref[:, 2 * D:3 * D]
    s3_ref[...] = lax.dot_general(a3, eh, (((1,), (1,)), ((), ())),
                                  preferred_element_type=jnp.float32)


def _tc_edge(edge_attr, W_r, attn_w):
    blk = 512
    grid = E // blk
    return pl.pallas_call(
        _tc_edge_body,
        grid=(grid,),
        in_specs=[
            pl.BlockSpec((blk, D), lambda i: (i, 0)),
            pl.BlockSpec((D, D), lambda i: (0, 0)),
            pl.BlockSpec((1, 3 * D), lambda i: (0, 0)),
        ],
        out_specs=[
            pl.BlockSpec((blk, DH), lambda i: (i, 0)),
            pl.BlockSpec((blk, DH), lambda i: (i, 0)),
            pl.BlockSpec((1, blk), lambda i: (0, i)),
        ],
        out_shape=[
            jax.ShapeDtypeStruct((E, DH), jnp.bfloat16),
            jax.ShapeDtypeStruct((E, DH), jnp.bfloat16),
            jax.ShapeDtypeStruct((1, E), jnp.float32),
        ],
    )(edge_attr, W_r, attn_w)


_SC_MESH = plsc.VectorSubcoreMesh(core_axis_name="c", subcore_axis_name="s")


@functools.partial(
    pl.kernel,
    mesh=_SC_MESH,
    compiler_params=pltpu.CompilerParams(needs_layout_passes=False,
                                         use_tc_tiling_on_sc=False),
    out_type=[
        jax.ShapeDtypeStruct((E,), jnp.float32),
        jax.ShapeDtypeStruct((NW * N_PAD,), jnp.float32),
    ],
    scratch_types=[
        pltpu.VMEM((N,), jnp.float32),
        pltpu.VMEM((N,), jnp.float32),
        pltpu.VMEM((N_PAD,), jnp.float32),
        pltpu.VMEM((2, P1_SUBS, CHUNK), jnp.int32),
        pltpu.VMEM((2, P1_SUBS, CHUNK), jnp.int32),
        pltpu.VMEM((2, P1_SUPER_E), jnp.float32),
        pltpu.VMEM((P1_SUPER_E,), jnp.float32),
        pltpu.SemaphoreType.DMA,
        pltpu.SemaphoreType.DMA,
    ],
)
def _sc_scalar_kernel(src_hbm, dst_hbm, s3_hbm, s1_hbm, s2_hbm,
                      ee_out, den_out,
                      s1_v, s2_v, den_v, src_v, dst_v, s3_v, ee_v, *psem):
    cid = lax.axis_index("c")
    sid = lax.axis_index("s")
    wid = cid * NS + sid

    zero16 = jnp.zeros((16,), jnp.float32)

    @pl.loop(0, N_PAD // 16)
    def _zden(r):
        den_v[pl.ds(r * 16, 16)] = zero16

    pltpu.sync_copy(s1_hbm, s1_v)
    pltpu.sync_copy(s2_hbm, s2_v)

    def prefetch(sc0, rb):
        pltpu.async_copy(src_hbm.at[pl.ds(sc0, P1_SUBS)], src_v.at[rb],
                         psem[rb])
        pltpu.async_copy(dst_hbm.at[pl.ds(sc0, P1_SUBS)], dst_v.at[rb],
                         psem[rb])
        pltpu.async_copy(s3_hbm.at[pl.ds(sc0 * CHUNK, P1_SUPER_E)],
                         s3_v.at[rb], psem[rb])

    def wait_prefetch(rb):
        pltpu.make_async_copy(src_hbm.at[pl.ds(0, P1_SUBS)], src_v.at[rb],
                              psem[rb]).wait()
        pltpu.make_async_copy(dst_hbm.at[pl.ds(0, P1_SUBS)], dst_v.at[rb],
                              psem[rb]).wait()
        pltpu.make_async_copy(s3_hbm.at[pl.ds(0, P1_SUPER_E)], s3_v.at[rb],
                              psem[rb]).wait()

    def scalar_groups(rb, jj):
        for g in range(CHUNK // 16):
            o = g * 16
            isrc = src_v[rb, jj, pl.ds(o, 16)]
            idst = dst_v[rb, jj, pl.ds(o, 16)]
            v = (plsc.load_gather(s1_v, [isrc])
                 + plsc.load_gather(s2_v, [idst])
                 + s3_v[rb, pl.ds(jj * CHUNK + o, 16)])
            v = jnp.where(v >= 0.0, v, 0.01 * v)
            ee = jnp.exp(v)
            ee_v[pl.ds(jj * CHUNK + o, 16)] = ee
            plsc.addupdate_scatter(den_v, [idst], ee)

    def compute_super(rb, sc0):
        @pl.loop(0, P1_SUBS)
        def _scalars(jj):
            scalar_groups(rb, jj)

        pltpu.sync_copy(ee_v, ee_out.at[pl.ds(sc0 * CHUNK, P1_SUPER_E)])

    start_chunk = wid * P1_BASE
    n_pairs = P1_SUPERS // 2
    prefetch(start_chunk, 0)

    @pl.loop(0, n_pairs)
    def _pair(p):
        sc_even = start_chunk + (2 * p) * P1_SUBS
        prefetch(sc_even + P1_SUBS, 1)
        wait_prefetch(0)
        compute_super(0, sc_even)

        @pl.when(p + 1 < n_pairs)
        def _():
            prefetch(sc_even + 2 * P1_SUBS, 0)

        wait_prefetch(1)
        compute_super(1, sc_even + P1_SUBS)

    @pl.when(wid < P1_EXTRA)
    def _tail():
        c = NW * P1_BASE + wid
        pltpu.sync_copy(src_hbm.at[pl.ds(c, 1)], src_v.at[0].at[pl.ds(0, 1)])
        pltpu.sync_copy(dst_hbm.at[pl.ds(c, 1)], dst_v.at[0].at[pl.ds(0, 1)])
        pltpu.sync_copy(s3_hbm.at[pl.ds(c * CHUNK, CHUNK)],
                        s3_v.at[0].at[pl.ds(0, CHUNK)])
        scalar_groups(0, 0)
        pltpu.sync_copy(ee_v.at[pl.ds(0, CHUNK)],
                        ee_out.at[pl.ds(c * CHUNK, CHUNK)])

    pltpu.sync_copy(den_v, den_out.at[pl.ds(wid * N_PAD, N_PAD)])


@functools.partial(
    pl.kernel,
    mesh=_SC_MESH,
    compiler_params=pltpu.CompilerParams(needs_layout_passes=False,
                                         use_tc_tiling_on_sc=False),
    out_type=jax.ShapeDtypeStruct((NC * N_PAD, DH), jnp.float32),
    scratch_types=[
        pltpu.VMEM((SUBS, CHUNK), jnp.int32),
        pltpu.VMEM((SUBS, CHUNK), jnp.int32),
        pltpu.VMEM((SUPER_E,), jnp.float32),
        pltpu.VMEM((NBH, CHUNK, DH), jnp.bfloat16),
        pltpu.VMEM((NBE, CHUNK, DH), jnp.bfloat16),
        pltpu.VMEM((NBH, CHUNK, DH), jnp.float32),
        pltpu.VMEM_SHARED((N_PAD, DH), jnp.float32),
    ] + [pltpu.SemaphoreType.DMA] * (2 * NBH + NBE),
)
def _sc_row_kernel(src_hbm, dst_hbm, ee_hbm,
                   hlo_hbm, hhi_hbm, ehlo_hbm, ehhi_hbm,
                   agg_out,
                   src_v, dst_v, ee_v, hrow_v, ehrow_v, m_v, agg_sp, *sems):
    gsem = sems[0:NBH]
    ssem = sems[NBH:2 * NBH]
    esem = sems[2 * NBH:2 * NBH + NBE]
    cid = lax.axis_index("c")
    sid = lax.axis_index("s")

    zero16 = jnp.zeros((16,), jnp.float32)
    izero16 = jnp.zeros((16,), jnp.int32)

    @pl.loop(0, CHUNK)
    def _zrows(r):
        for b in range(NBH):
            for sg in range(DH // 16):
                m_v[b, r, pl.ds(sg * 16, 16)] = zero16

    @pl.loop(0, SUBS)
    def _zidx(r):
        for sg in range(CHUNK // 16):
            dst_v[r, pl.ds(sg * 16, 16)] = izero16

    row0 = sid * ROWS_PER_TILE
    for k in range(ROWS_PER_TILE // CHUNK):
        pltpu.sync_copy(m_v.at[0], agg_sp.at[pl.ds(row0 + k * CHUNK, CHUNK)])
    rem = ROWS_PER_TILE % CHUNK
    if rem:
        pltpu.sync_copy(m_v.at[0].at[pl.ds(0, rem)],
                        agg_sp.at[pl.ds(row0 + (ROWS_PER_TILE // CHUNK) * CHUNK,
                                        rem)])

    plsc.subcore_barrier()

    for b in range(NBH):
        pltpu.async_copy(m_v.at[b], agg_sp.at[dst_v.at[0]], ssem[b],
                         add=True)

    start_chunk = sid * BASE_CHUNKS
    is_lo = cid == 0

    def start_gather(j, bh):
        @pl.when(is_lo)
        def _():
            pltpu.async_copy(hlo_hbm.at[src_v.at[j]], hrow_v.at[bh], gsem[bh])

        @pl.when(jnp.logical_not(is_lo))
        def _():
            pltpu.async_copy(hhi_hbm.at[src_v.at[j]], hrow_v.at[bh], gsem[bh])

    def start_eh(chunk_idx, be):
        @pl.when(is_lo)
        def _():
            pltpu.async_copy(ehlo_hbm.at[pl.ds(chunk_idx * CHUNK, CHUNK)],
                             ehrow_v.at[be], esem[be])

        @pl.when(jnp.logical_not(is_lo))
        def _():
            pltpu.async_copy(ehhi_hbm.at[pl.ds(chunk_idx * CHUNK, CHUNK)],
                             ehrow_v.at[be], esem[be])

    def wait_gather(bh):
        pltpu.make_async_copy(hlo_hbm.at[pl.ds(0, CHUNK)], hrow_v.at[bh],
                              gsem[bh]).wait()

    def wait_eh(be):
        pltpu.make_async_copy(ehlo_hbm.at[pl.ds(0, CHUNK)], ehrow_v.at[be],
                              esem[be]).wait()

    def wait_scatter(bh):
        pltpu.make_async_copy(agg_out.at[pl.ds(0, CHUNK)], m_v.at[bh],
                              ssem[bh]).wait()

    def do_rows(j, bh, be):
        @pl.loop(0, CHUNK // 16)
        def group_body(g):
            ee_g = ee_v[pl.ds(j * CHUNK + g * 16, 16)]
            for l in range(16):
                i = g * 16 + l
                s = ee_g[l]
                for sg in range(DH // 32):
                    hp = hrow_v[bh, i, pl.ds(sg * 32, 32)]
                    ep = ehrow_v[be, i, pl.ds(sg * 32, 32)]
                    ha, hb = plsc.unpack(hp, format=plsc.PackFormat.INTERLEAVED)
                    ea, eb = plsc.unpack(ep, format=plsc.PackFormat.INTERLEAVED)
                    m_v[bh, i, pl.ds(sg * 32, 16)] = s * (ha + ea)
                    m_v[bh, i, pl.ds(sg * 32 + 16, 16)] = s * (hb + eb)

        pltpu.async_copy(m_v.at[bh], agg_sp.at[dst_v.at[j]], ssem[bh],
                         add=True)

    @pl.loop(0, SUPERS)
    def _super(s):
        sc0 = start_chunk + s * SUBS
        pltpu.sync_copy(src_hbm.at[pl.ds(sc0, SUBS)], src_v)
        pltpu.sync_copy(dst_hbm.at[pl.ds(sc0, SUBS)], dst_v)
        pltpu.sync_copy(ee_hbm.at[pl.ds(sc0 * CHUNK, SUPER_E)], ee_v)

        wait_scatter(0)
        start_gather(0, 0)
        wait_scatter(1)
        start_gather(1, 1)
        start_eh(sc0 + 0, 0)

        for j in range(SUBS):
            bh = j % NBH
            be = j % NBE
            wait_gather(bh)
            wait_eh(be)
            do_rows(j, bh, be)
            if j + 1 < SUBS:
                start_eh(sc0 + j + 1, (j + 1) % NBE)
            if j + 2 < SUBS:
                bm = (j + 2) % NBH
                wait_scatter(bm)
                start_gather(j + 2, bm)

    @pl.when(sid < EXTRA_CHUNKS)
    def _tail():
        c = NS * BASE_CHUNKS + sid
        wait_scatter(0)
        pltpu.sync_copy(src_hbm.at[pl.ds(c, 1)], src_v.at[pl.ds(0, 1)])
        pltpu.sync_copy(dst_hbm.at[pl.ds(c, 1)], dst_v.at[pl.ds(0, 1)])
        pltpu.sync_copy(ee_hbm.at[pl.ds(c * CHUNK, CHUNK)],
                        ee_v.at[pl.ds(0, CHUNK)])
        start_gather(0, 0)
        start_eh(c, 0)
        wait_gather(0)
        wait_eh(0)
        do_rows(0, 0, 0)
        wait_scatter(0)

    @pl.when(sid >= EXTRA_CHUNKS)
    def _():
        wait_scatter(0)
    for b in range(1, NBH):
        wait_scatter(b)

    plsc.subcore_barrier()

    pltpu.sync_copy(agg_sp.at[pl.ds(row0, ROWS_PER_TILE)],
                    agg_out.at[pl.ds(cid * N_PAD + row0, ROWS_PER_TILE)])


def _tc_final_body(agglo_ref, agghi_ref, den_ref, hl_ref, out_ref):
    a = jnp.concatenate([agglo_ref[...], agghi_ref[...]], axis=1)
    d = jnp.sum(den_ref[...], axis=1, keepdims=True)
    safe = d > 0.0
    dd = jnp.where(safe, d, 1.0)
    r = jnp.where(safe, a / dd, 0.0)
    out_ref[...] = jnp.maximum(r + hl_ref[...], 0.0)


def _tc_final(agg_lo, agg_hi, denT, hl):
    blk = 400
    grid = N // blk
    return pl.pallas_call(
        _tc_final_body,
        grid=(grid,),
        in_specs=[
            pl.BlockSpec((blk, DH), lambda i: (i, 0)),
            pl.BlockSpec((blk, DH), lambda i: (i, 0)),
            pl.BlockSpec((blk, NW), lambda i: (i, 0)),
            pl.BlockSpec((blk, D), lambda i: (i, 0)),
        ],
        out_specs=pl.BlockSpec((blk, D), lambda i: (i, 0)),
        out_shape=jax.ShapeDtypeStruct((N, D), jnp.float32),
    )(agg_lo, agg_hi, denT, hl)


def kernel(x, edge_index, edge_attr, W_fc, W_r, attn_w, loop_w):
    x_pad = jnp.pad(x, ((0, N_TC - N), (0, 0)))
    h_lo, h_hi, hl_pad, s12 = _tc_node(x_pad, W_fc, loop_w, attn_w)
    eh_lo, eh_hi, s3p = _tc_edge(edge_attr, W_r, attn_w)

    s1 = s12[0, :N]
    s2 = s12[1, :N]
    s3 = s3p[0]
    src = edge_index[0].astype(jnp.int32).reshape(NCHUNKS, CHUNK)
    dst = edge_index[1].astype(jnp.int32).reshape(NCHUNKS, CHUNK)

    ee, denp = _sc_scalar_kernel(src, dst, s3, s1, s2)
    agg2 = _sc_row_kernel(src, dst, ee, h_lo[:N], h_hi[:N], eh_lo, eh_hi)
    inv = np.arange(DH).reshape(DH // 32, 2, 16).transpose(0, 2, 1).reshape(DH)
    agg_lo = jnp.take(agg2[:N], inv, axis=1)
    agg_hi = jnp.take(agg2[N_PAD:N_PAD + N], inv, axis=1)
    denT = denp.reshape(NW, N_PAD).T[:N]
    return _tc_final(agg_lo, agg_hi, denT, hl_pad[:N])


# speedup vs baseline: 1.1378x; 1.1378x over previous
import functools

import jax
import jax.numpy as jnp
from jax import lax
from jax.experimental import pallas as pl
from jax.experimental.pallas import tpu as pltpu
from jax.experimental.pallas import tpu_sc as plsc

N = 10000
E = 320000
D = 128
DH = D // 2

N_TC = 10240
N_PAD = 10112
NC = 2
NS = 16
CHUNK = 128
NCHUNKS = E // CHUNK
ROWS_PER_TILE = N_PAD // NS

NW = NC * NS
P1_BASE = NCHUNKS // NW
P1_EXTRA = NCHUNKS % NW
P1_SUBS = 13
P1_SUPERS = P1_BASE // P1_SUBS
P1_SUPER_E = P1_SUBS * CHUNK

BASE_CHUNKS = NCHUNKS // NS
EXTRA_CHUNKS = NCHUNKS % NS
SUBS = 12
SUPERS = BASE_CHUNKS // SUBS
SUPER_E = SUBS * CHUNK
NBH = 3
NBE = 2


def _tc_node_body(x_ref, wfc_ref, lw_ref, aw_ref,
                  hlo_ref, hhi_ref, hl_ref, s12_ref):
    x = x_ref[...]
    h = lax.dot_general(x, wfc_ref[...], (((1,), (1,)), ((), ())),
                        preferred_element_type=jnp.float32)
    hlo_ref[...] = h[:, :DH]
    hhi_ref[...] = h[:, DH:]
    hl_ref[...] = lax.dot_general(h, lw_ref[...], (((1,), (0,)), ((), ())),
                                  preferred_element_type=jnp.float32)
    a1 = aw_ref[:, 0:D]
    a2 = aw_ref[:, D:2 * D]
    s1 = lax.dot_general(a1, h, (((1,), (1,)), ((), ())),
                         preferred_element_type=jnp.float32)
    s2 = lax.dot_general(a2, h, (((1,), (1,)), ((), ())),
                         preferred_element_type=jnp.float32)
    s12_ref[...] = jnp.concatenate(
        [s1, s2, jnp.zeros((6, s1.shape[1]), jnp.float32)], axis=0)


def _tc_node(x_pad, W_fc, loop_w, attn_w):
    blk = 512
    grid = N_TC // blk
    return pl.pallas_call(
        _tc_node_body,
        grid=(grid,),
        in_specs=[
            pl.BlockSpec((blk, D), lambda i: (i, 0)),
            pl.BlockSpec((D, D), lambda i: (0, 0)),
            pl.BlockSpec((D, D), lambda i: (0, 0)),
            pl.BlockSpec((1, 3 * D), lambda i: (0, 0)),
        ],
        out_specs=[
            pl.BlockSpec((blk, DH), lambda i: (i, 0)),
            pl.BlockSpec((blk, DH), lambda i: (i, 0)),
            pl.BlockSpec((blk, D), lambda i: (i, 0)),
            pl.BlockSpec((8, blk), lambda i: (0, i)),
        ],
        out_shape=[
            jax.ShapeDtypeStruct((N_TC, DH), jnp.float32),
            jax.ShapeDtypeStruct((N_TC, DH), jnp.float32),
            jax.ShapeDtypeStruct((N_TC, D), jnp.float32),
            jax.ShapeDtypeStruct((8, N_TC), jnp.float32),
        ],
    )(x_pad, W_fc, loop_w, attn_w)


def _tc_edge_body(ea_ref, wr_ref, aw_ref, ehlo_ref, ehhi_ref, s3_ref):
    eh = lax.dot_general(ea_ref[...], wr_ref[...], (((1,), (1,)), ((), ())),
                         preferred_element_type=jnp.float32)
    ehlo_ref[...] = eh[:, :DH]
    ehhi_ref[...] = eh[:, DH:]
    a3 = aw_ref[:, 2 * D:3 * D]
    s3_ref[...] = lax.dot_general(a3, eh, (((1,), (1,)), ((), ())),
                                  preferred_element_type=jnp.float32)


def _tc_edge(edge_attr, W_r, attn_w):
    blk = 512
    grid = E // blk
    return pl.pallas_call(
        _tc_edge_body,
        grid=(grid,),
        in_specs=[
            pl.BlockSpec((blk, D), lambda i: (i, 0)),
            pl.BlockSpec((D, D), lambda i: (0, 0)),
            pl.BlockSpec((1, 3 * D), lambda i: (0, 0)),
        ],
        out_specs=[
            pl.BlockSpec((blk, DH), lambda i: (i, 0)),
            pl.BlockSpec((blk, DH), lambda i: (i, 0)),
            pl.BlockSpec((1, blk), lambda i: (0, i)),
        ],
        out_shape=[
            jax.ShapeDtypeStruct((E, DH), jnp.float32),
            jax.ShapeDtypeStruct((E, DH), jnp.float32),
            jax.ShapeDtypeStruct((1, E), jnp.float32),
        ],
    )(edge_attr, W_r, attn_w)


_SC_MESH = plsc.VectorSubcoreMesh(core_axis_name="c", subcore_axis_name="s")


@functools.partial(
    pl.kernel,
    mesh=_SC_MESH,
    compiler_params=pltpu.CompilerParams(needs_layout_passes=False,
                                         use_tc_tiling_on_sc=False),
    out_type=[
        jax.ShapeDtypeStruct((E,), jnp.float32),
        jax.ShapeDtypeStruct((NW * N_PAD,), jnp.float32),
    ],
    scratch_types=[
        pltpu.VMEM((N,), jnp.float32),
        pltpu.VMEM((N,), jnp.float32),
        pltpu.VMEM((N_PAD,), jnp.float32),
        pltpu.VMEM((2, P1_SUBS, CHUNK), jnp.int32),
        pltpu.VMEM((2, P1_SUBS, CHUNK), jnp.int32),
        pltpu.VMEM((2, P1_SUPER_E), jnp.float32),
        pltpu.VMEM((P1_SUPER_E,), jnp.float32),
        pltpu.SemaphoreType.DMA,
        pltpu.SemaphoreType.DMA,
    ],
)
def _sc_scalar_kernel(src_hbm, dst_hbm, s3_hbm, s1_hbm, s2_hbm,
                      ee_out, den_out,
                      s1_v, s2_v, den_v, src_v, dst_v, s3_v, ee_v, *psem):
    cid = lax.axis_index("c")
    sid = lax.axis_index("s")
    wid = cid * NS + sid

    zero16 = jnp.zeros((16,), jnp.float32)

    @pl.loop(0, N_PAD // 16)
    def _zden(r):
        den_v[pl.ds(r * 16, 16)] = zero16

    pltpu.sync_copy(s1_hbm, s1_v)
    pltpu.sync_copy(s2_hbm, s2_v)

    def prefetch(sc0, rb):
        pltpu.async_copy(src_hbm.at[pl.ds(sc0, P1_SUBS)], src_v.at[rb],
                         psem[rb])
        pltpu.async_copy(dst_hbm.at[pl.ds(sc0, P1_SUBS)], dst_v.at[rb],
                         psem[rb])
        pltpu.async_copy(s3_hbm.at[pl.ds(sc0 * CHUNK, P1_SUPER_E)],
                         s3_v.at[rb], psem[rb])

    def wait_prefetch(rb):
        pltpu.make_async_copy(src_hbm.at[pl.ds(0, P1_SUBS)], src_v.at[rb],
                              psem[rb]).wait()
        pltpu.make_async_copy(dst_hbm.at[pl.ds(0, P1_SUBS)], dst_v.at[rb],
                              psem[rb]).wait()
        pltpu.make_async_copy(s3_hbm.at[pl.ds(0, P1_SUPER_E)], s3_v.at[rb],
                              psem[rb]).wait()

    def scalar_groups(rb, jj):
        for g in range(CHUNK // 16):
            o = g * 16
            isrc = src_v[rb, jj, pl.ds(o, 16)]
            idst = dst_v[rb, jj, pl.ds(o, 16)]
            v = (plsc.load_gather(s1_v, [isrc])
                 + plsc.load_gather(s2_v, [idst])
                 + s3_v[rb, pl.ds(jj * CHUNK + o, 16)])
            v = jnp.where(v >= 0.0, v, 0.01 * v)
            ee = jnp.exp(v)
            ee_v[pl.ds(jj * CHUNK + o, 16)] = ee
            plsc.addupdate_scatter(den_v, [idst], ee)

    def compute_super(rb, sc0):
        @pl.loop(0, P1_SUBS)
        def _scalars(jj):
            scalar_groups(rb, jj)

        pltpu.sync_copy(ee_v, ee_out.at[pl.ds(sc0 * CHUNK, P1_SUPER_E)])

    start_chunk = wid * P1_BASE
    n_pairs = P1_SUPERS // 2
    prefetch(start_chunk, 0)

    @pl.loop(0, n_pairs)
    def _pair(p):
        sc_even = start_chunk + (2 * p) * P1_SUBS
        prefetch(sc_even + P1_SUBS, 1)
        wait_prefetch(0)
        compute_super(0, sc_even)

        @pl.when(p + 1 < n_pairs)
        def _():
            prefetch(sc_even + 2 * P1_SUBS, 0)

        wait_prefetch(1)
        compute_super(1, sc_even + P1_SUBS)

    @pl.when(wid < P1_EXTRA)
    def _tail():
        c = NW * P1_BASE + wid
        pltpu.sync_copy(src_hbm.at[pl.ds(c, 1)], src_v.at[0].at[pl.ds(0, 1)])
        pltpu.sync_copy(dst_hbm.at[pl.ds(c, 1)], dst_v.at[0].at[pl.ds(0, 1)])
        pltpu.sync_copy(s3_hbm.at[pl.ds(c * CHUNK, CHUNK)],
                        s3_v.at[0].at[pl.ds(0, CHUNK)])
        scalar_groups(0, 0)
        pltpu.sync_copy(ee_v.at[pl.ds(0, CHUNK)],
                        ee_out.at[pl.ds(c * CHUNK, CHUNK)])

    pltpu.sync_copy(den_v, den_out.at[pl.ds(wid * N_PAD, N_PAD)])


@functools.partial(
    pl.kernel,
    mesh=_SC_MESH,
    compiler_params=pltpu.CompilerParams(needs_layout_passes=False,
                                         use_tc_tiling_on_sc=False),
    out_type=jax.ShapeDtypeStruct((NC * N_PAD, DH), jnp.float32),
    scratch_types=[
        pltpu.VMEM((SUBS, CHUNK), jnp.int32),
        pltpu.VMEM((SUBS, CHUNK), jnp.int32),
        pltpu.VMEM((SUPER_E,), jnp.float32),
        pltpu.VMEM((NBH, CHUNK, DH), jnp.float32),
        pltpu.VMEM((NBE, CHUNK, DH), jnp.float32),
        pltpu.VMEM_SHARED((N_PAD, DH), jnp.float32),
    ] + [pltpu.SemaphoreType.DMA] * (2 * NBH + NBE),
)
def _sc_row_kernel(src_hbm, dst_hbm, ee_hbm,
                   hlo_hbm, hhi_hbm, ehlo_hbm, ehhi_hbm,
                   agg_out,
                   src_v, dst_v, ee_v, hrow_v, ehrow_v, agg_sp, *sems):
    gsem = sems[0:NBH]
    ssem = sems[NBH:2 * NBH]
    esem = sems[2 * NBH:2 * NBH + NBE]
    cid = lax.axis_index("c")
    sid = lax.axis_index("s")

    zero16 = jnp.zeros((16,), jnp.float32)
    izero16 = jnp.zeros((16,), jnp.int32)

    @pl.loop(0, CHUNK)
    def _zrows(r):
        for b in range(NBH):
            for sg in range(DH // 16):
                hrow_v[b, r, pl.ds(sg * 16, 16)] = zero16

    @pl.loop(0, SUBS)
    def _zidx(r):
        for sg in range(CHUNK // 16):
            dst_v[r, pl.ds(sg * 16, 16)] = izero16

    row0 = sid * ROWS_PER_TILE
    for k in range(ROWS_PER_TILE // CHUNK):
        pltpu.sync_copy(hrow_v.at[0], agg_sp.at[pl.ds(row0 + k * CHUNK, CHUNK)])
    rem = ROWS_PER_TILE % CHUNK
    if rem:
        pltpu.sync_copy(hrow_v.at[0].at[pl.ds(0, rem)],
                        agg_sp.at[pl.ds(row0 + (ROWS_PER_TILE // CHUNK) * CHUNK,
                                        rem)])

    plsc.subcore_barrier()

    for b in range(NBH):
        pltpu.async_copy(hrow_v.at[b], agg_sp.at[dst_v.at[0]], ssem[b],
                         add=True)

    start_chunk = sid * BASE_CHUNKS
    is_lo = cid == 0

    def start_gather(j, bh):
        @pl.when(is_lo)
        def _():
            pltpu.async_copy(hlo_hbm.at[src_v.at[j]], hrow_v.at[bh], gsem[bh])

        @pl.when(jnp.logical_not(is_lo))
        def _():
            pltpu.async_copy(hhi_hbm.at[src_v.at[j]], hrow_v.at[bh], gsem[bh])

    def start_eh(chunk_idx, be):
        @pl.when(is_lo)
        def _():
            pltpu.async_copy(ehlo_hbm.at[pl.ds(chunk_idx * CHUNK, CHUNK)],
                             ehrow_v.at[be], esem[be])

        @pl.when(jnp.logical_not(is_lo))
        def _():
            pltpu.async_copy(ehhi_hbm.at[pl.ds(chunk_idx * CHUNK, CHUNK)],
                             ehrow_v.at[be], esem[be])

    def wait_gather(bh):
        pltpu.make_async_copy(hlo_hbm.at[pl.ds(0, CHUNK)], hrow_v.at[bh],
                              gsem[bh]).wait()

    def wait_eh(be):
        pltpu.make_async_copy(ehlo_hbm.at[pl.ds(0, CHUNK)], ehrow_v.at[be],
                              esem[be]).wait()

    def wait_scatter(bh):
        pltpu.make_async_copy(hlo_hbm.at[pl.ds(0, CHUNK)], hrow_v.at[bh],
                              ssem[bh]).wait()

    def do_rows(j, bh, be):
        @pl.loop(0, CHUNK // 16)
        def group_body(g):
            ee_g = ee_v[pl.ds(j * CHUNK + g * 16, 16)]
            for l in range(16):
                i = g * 16 + l
                s = ee_g[l]
                for sg in range(DH // 16):
                    hseg = hrow_v[bh, i, pl.ds(sg * 16, 16)]
                    eseg = ehrow_v[be, i, pl.ds(sg * 16, 16)]
                    hrow_v[bh, i, pl.ds(sg * 16, 16)] = s * (hseg + eseg)

        pltpu.async_copy(hrow_v.at[bh], agg_sp.at[dst_v.at[j]], ssem[bh],
                         add=True)

    @pl.loop(0, SUPERS)
    def _super(s):
        sc0 = start_chunk + s * SUBS
        pltpu.sync_copy(src_hbm.at[pl.ds(sc0, SUBS)], src_v)
        pltpu.sync_copy(dst_hbm.at[pl.ds(sc0, SUBS)], dst_v)
        pltpu.sync_copy(ee_hbm.at[pl.ds(sc0 * CHUNK, SUPER_E)], ee_v)

        wait_scatter(0)
        start_gather(0, 0)
        wait_scatter(1)
        start_gather(1, 1)
        start_eh(sc0 + 0, 0)

        for j in range(SUBS):
            bh = j % NBH
            be = j % NBE
            wait_gather(bh)
            wait_eh(be)
            do_rows(j, bh, be)
            if j + 1 < SUBS:
                start_eh(sc0 + j + 1, (j + 1) % NBE)
            if j + 2 < SUBS:
                bm = (j + 2) % NBH
                wait_scatter(bm)
                start_gather(j + 2, bm)

    @pl.when(sid < EXTRA_CHUNKS)
    def _tail():
        c = NS * BASE_CHUNKS + sid
        wait_scatter(0)
        pltpu.sync_copy(src_hbm.at[pl.ds(c, 1)], src_v.at[pl.ds(0, 1)])
        pltpu.sync_copy(dst_hbm.at[pl.ds(c, 1)], dst_v.at[pl.ds(0, 1)])
        pltpu.sync_copy(ee_hbm.at[pl.ds(c * CHUNK, CHUNK)],
                        ee_v.at[pl.ds(0, CHUNK)])
        start_gather(0, 0)
        start_eh(c, 0)
        wait_gather(0)
        wait_eh(0)
        do_rows(0, 0, 0)
        wait_scatter(0)

    @pl.when(sid >= EXTRA_CHUNKS)
    def _():
        wait_scatter(0)
    for b in range(1, NBH):
        wait_scatter(b)

    plsc.subcore_barrier()

    pltpu.sync_copy(agg_sp.at[pl.ds(row0, ROWS_PER_TILE)],
                    agg_out.at[pl.ds(cid * N_PAD + row0, ROWS_PER_TILE)])


def _tc_final_body(agglo_ref, agghi_ref, den_ref, hl_ref, out_ref):
    a = jnp.concatenate([agglo_ref[...], agghi_ref[...]], axis=1)
    d = jnp.sum(den_ref[...], axis=1, keepdims=True)
    safe = d > 0.0
    dd = jnp.where(safe, d, 1.0)
    r = jnp.where(safe, a / dd, 0.0)
    out_ref[...] = jnp.maximum(r + hl_ref[...], 0.0)


def _tc_final(agg_lo, agg_hi, denT, hl):
    blk = 400
    grid = N // blk
    return pl.pallas_call(
        _tc_final_body,
        grid=(grid,),
        in_specs=[
            pl.BlockSpec((blk, DH), lambda i: (i, 0)),
            pl.BlockSpec((blk, DH), lambda i: (i, 0)),
            pl.BlockSpec((blk, NW), lambda i: (i, 0)),
            pl.BlockSpec((blk, D), lambda i: (i, 0)),
        ],
        out_specs=pl.BlockSpec((blk, D), lambda i: (i, 0)),
        out_shape=jax.ShapeDtypeStruct((N, D), jnp.float32),
    )(agg_lo, agg_hi, denT, hl)


def kernel(x, edge_index, edge_attr, W_fc, W_r, attn_w, loop_w):
    x_pad = jnp.pad(x, ((0, N_TC - N), (0, 0)))
    h_lo, h_hi, hl_pad, s12 = _tc_node(x_pad, W_fc, loop_w, attn_w)
    eh_lo, eh_hi, s3p = _tc_edge(edge_attr, W_r, attn_w)

    s1 = s12[0, :N]
    s2 = s12[1, :N]
    s3 = s3p[0]
    src = edge_index[0].astype(jnp.int32).reshape(NCHUNKS, CHUNK)
    dst = edge_index[1].astype(jnp.int32).reshape(NCHUNKS, CHUNK)

    ee, denp = _sc_scalar_kernel(src, dst, s3, s1, s2)
    agg2 = _sc_row_kernel(src, dst, ee, h_lo[:N], h_hi[:N], eh_lo, eh_hi)
    agg_lo = agg2[:N]
    agg_hi = agg2[N_PAD:N_PAD + N]
    denT = denp.reshape(NW, N_PAD).T[:N]
    return _tc_final(agg_lo, agg_hi, denT, hl_pad[:N])
